# baseline (speedup 1.0000x reference)
import functools

import jax
import jax.numpy as jnp
from jax import lax
from jax.experimental import pallas as pl
from jax.experimental.pallas import tpu as pltpu
from jax.experimental.pallas import tpu_sc as plsc

EMBED = 64
CH = 200


def _sc_gather(table, x):
    bs, seqk = x.shape
    info = plsc.get_sparse_core_info()
    nc, ns = info.num_cores, info.num_subcores
    nw = nc * ns
    b_per_w = bs // nw
    assert bs % nw == 0 and seqk % 2 == 0
    half = seqk // 2

    mesh = plsc.VectorSubcoreMesh(core_axis_name="c", subcore_axis_name="s")

    @functools.partial(
        pl.kernel,
        mesh=mesh,
        out_type=jax.ShapeDtypeStruct((bs * seqk, EMBED), jnp.float32),
        scratch_types=[
            pltpu.VMEM((seqk,), jnp.int32),
            pltpu.VMEM((seqk, EMBED), jnp.float32),
            pltpu.SemaphoreType.DMA,
        ],
        compiler_params=pltpu.CompilerParams(use_tc_tiling_on_sc=False),
    )
    def gather_kernel(table_hbm, x_hbm, out_hbm, idx_v, rows_v, sem):
        wid = lax.axis_index("s") * nc + lax.axis_index("c")
        base = wid * b_per_w

        def body(i, carry):
            b = base + i
            pltpu.sync_copy(x_hbm.at[b], idx_v)
            pltpu.async_copy(table_hbm.at[idx_v], rows_v, sem).wait()
            pltpu.sync_copy(rows_v, out_hbm.at[pl.ds(b * seqk, seqk)])
            return carry

        lax.fori_loop(0, b_per_w, body, 0)

    return gather_kernel(table, x)


def _tc_project(g2, wfull, bias2, n_tok):
    m5 = g2.shape[0]
    bm5 = 5120
    assert m5 % bm5 == 0
    grid = m5 // bm5
    bp = bm5 // 5

    def body(g_ref, w_ref, b_ref, o_ref):
        xg = g_ref[...].reshape(bp, 5 * 128)
        acc = jnp.dot(xg, w_ref[...], preferred_element_type=jnp.float32)
        o_ref[...] = jnp.maximum(acc + b_ref[...], 0.0)

    return pl.pallas_call(
        body,
        grid=(grid,),
        in_specs=[
            pl.BlockSpec((bm5, 128), lambda i: (i, 0)),
            pl.BlockSpec((5 * 128, 128), lambda i: (0, 0)),
            pl.BlockSpec((1, 128), lambda i: (0, 0)),
        ],
        out_specs=pl.BlockSpec((bp, 128), lambda i: (i, 0)),
        out_shape=jax.ShapeDtypeStruct((n_tok // 2, 128), jnp.float32),
        compiler_params=pltpu.CompilerParams(
            dimension_semantics=("arbitrary",),
        ),
    )(g2, wfull, bias2)


NSLICE = 2


def kernel(x, table, W, b):
    bs, seq, k = x.shape
    ke = k * EMBED
    v = table.shape[0]
    tablep = jnp.pad(table, ((0, 0), (0, EMBED))).reshape(2 * v, EMBED)
    x2 = (x * 2).reshape(bs, seq * k)
    z = jnp.zeros((ke, EMBED), jnp.float32)
    wfull = jnp.concatenate(
        [jnp.concatenate([W, z], axis=1), jnp.concatenate([z, W], axis=1)],
        axis=0,
    )
    bias2 = jnp.concatenate([b, b]).reshape(1, 2 * EMBED)

    bsl = bs // NSLICE
    outs = []
    for s in range(NSLICE):
        xs = lax.slice_in_dim(x2, s * bsl, (s + 1) * bsl, axis=0)
        g = _sc_gather(tablep, xs)
        g2 = g.reshape(bsl * seq * k // 2, 2 * EMBED)
        outs.append(_tc_project(g2, wfull, bias2, bsl * seq))
    out2 = jnp.concatenate(outs, axis=0)
    return out2.reshape(bs, seq, EMBED)


# speedup vs baseline: 1.0098x; 1.0098x over previous
import functools

import jax
import jax.numpy as jnp
from jax import lax
from jax.experimental import pallas as pl
from jax.experimental.pallas import tpu as pltpu
from jax.experimental.pallas import tpu_sc as plsc

EMBED = 64
CH = 200


def _sc_gather(table, x):
    bs, seqk = x.shape
    info = plsc.get_sparse_core_info()
    nc, ns = info.num_cores, info.num_subcores
    nw = nc * ns
    b_per_w = bs // nw
    assert bs % nw == 0 and seqk % 2 == 0
    half = seqk // 2

    mesh = plsc.VectorSubcoreMesh(core_axis_name="c", subcore_axis_name="s")

    @functools.partial(
        pl.kernel,
        mesh=mesh,
        out_type=jax.ShapeDtypeStruct((bs * seqk, EMBED), jnp.float32),
        scratch_types=[
            pltpu.VMEM((seqk,), jnp.int32),
            pltpu.VMEM((seqk, EMBED), jnp.float32),
            pltpu.SemaphoreType.DMA,
        ],
        compiler_params=pltpu.CompilerParams(use_tc_tiling_on_sc=False),
    )
    def gather_kernel(table_hbm, x_hbm, out_hbm, idx_v, rows_v, sem):
        wid = lax.axis_index("s") * nc + lax.axis_index("c")
        base = wid * b_per_w

        def body(i, carry):
            b = base + i
            pltpu.sync_copy(x_hbm.at[b], idx_v)
            pltpu.async_copy(table_hbm.at[idx_v], rows_v, sem).wait()
            pltpu.sync_copy(rows_v, out_hbm.at[pl.ds(b * seqk, seqk)])
            return carry

        lax.fori_loop(0, b_per_w, body, 0)

    return gather_kernel(table, x)


def _tc_project(g2, wfull, bias2, n_tok):
    m5 = g2.shape[0]
    bm5 = 12800
    assert m5 % bm5 == 0
    grid = m5 // bm5
    bp = bm5 // 5

    def body(g_ref, w_ref, b_ref, o_ref):
        xg = g_ref[...].reshape(bp, 5 * 128)
        acc = jnp.dot(xg, w_ref[...], preferred_element_type=jnp.float32)
        o_ref[...] = jnp.maximum(acc + b_ref[...], 0.0)

    return pl.pallas_call(
        body,
        grid=(grid,),
        in_specs=[
            pl.BlockSpec((bm5, 128), lambda i: (i, 0)),
            pl.BlockSpec((5 * 128, 128), lambda i: (0, 0)),
            pl.BlockSpec((1, 128), lambda i: (0, 0)),
        ],
        out_specs=pl.BlockSpec((bp, 128), lambda i: (i, 0)),
        out_shape=jax.ShapeDtypeStruct((n_tok // 2, 128), jnp.float32),
        compiler_params=pltpu.CompilerParams(
            dimension_semantics=("arbitrary",),
        ),
    )(g2, wfull, bias2)


NSLICE = 4


def kernel(x, table, W, b):
    bs, seq, k = x.shape
    ke = k * EMBED
    v = table.shape[0]
    tablep = jnp.pad(table, ((0, 0), (0, EMBED))).reshape(2 * v, EMBED)
    x2 = (x * 2).reshape(bs, seq * k)
    z = jnp.zeros((ke, EMBED), jnp.float32)
    wfull = jnp.concatenate(
        [jnp.concatenate([W, z], axis=1), jnp.concatenate([z, W], axis=1)],
        axis=0,
    )
    bias2 = jnp.concatenate([b, b]).reshape(1, 2 * EMBED)

    bsl = bs // NSLICE
    outs = []
    for s in range(NSLICE):
        xs = lax.slice_in_dim(x2, s * bsl, (s + 1) * bsl, axis=0)
        g = _sc_gather(tablep, xs)
        g2 = g.reshape(bsl * seq * k // 2, 2 * EMBED)
        outs.append(_tc_project(g2, wfull, bias2, bsl * seq))
    out2 = jnp.concatenate(outs, axis=0)
    return out2.reshape(bs, seq, EMBED)


# speedup vs baseline: 1.0104x; 1.0006x over previous
import functools

import jax
import jax.numpy as jnp
from jax import lax
from jax.experimental import pallas as pl
from jax.experimental.pallas import tpu as pltpu
from jax.experimental.pallas import tpu_sc as plsc

EMBED = 64


def _sc_gather(table, x):
    bs, seqk = x.shape
    info = plsc.get_sparse_core_info()
    nc, ns = info.num_cores, info.num_subcores
    nw = nc * ns
    b_per_w = bs // nw
    assert bs % nw == 0 and seqk % 2 == 0

    mesh = plsc.VectorSubcoreMesh(core_axis_name="c", subcore_axis_name="s")

    @functools.partial(
        pl.kernel,
        mesh=mesh,
        out_type=jax.ShapeDtypeStruct((bs * seqk, EMBED), jnp.float32),
        scratch_types=[
            pltpu.VMEM((seqk,), jnp.int32),
            pltpu.VMEM((seqk, EMBED), jnp.float32),
            pltpu.SemaphoreType.DMA,
        ],
        compiler_params=pltpu.CompilerParams(use_tc_tiling_on_sc=False),
    )
    def gather_kernel(table_hbm, x_hbm, out_hbm, idx_v, rows_v, sem):
        wid = lax.axis_index("s") * nc + lax.axis_index("c")
        base = wid * b_per_w

        def body(i, carry):
            b = base + i
            pltpu.sync_copy(x_hbm.at[b], idx_v)
            pltpu.async_copy(table_hbm.at[idx_v], rows_v, sem).wait()
            pltpu.sync_copy(rows_v, out_hbm.at[pl.ds(b * seqk, seqk)])
            return carry

        lax.fori_loop(0, b_per_w, body, 0)

    return gather_kernel(table, x)


def _tc_project(g2, wfull, bias2, n_tok):
    m5 = g2.shape[0]
    bm5 = 12800
    assert m5 % bm5 == 0
    grid = m5 // bm5
    bp = bm5 // 5

    def body(g_ref, w_ref, b_ref, o_ref):
        xg = g_ref[...].reshape(bp, 5 * 128)
        acc = jnp.dot(xg, w_ref[...], preferred_element_type=jnp.float32)
        o_ref[...] = jnp.maximum(acc + b_ref[...], 0.0)

    return pl.pallas_call(
        body,
        grid=(grid,),
        in_specs=[
            pl.BlockSpec((bm5, 128), lambda i: (i, 0)),
            pl.BlockSpec((5 * 128, 128), lambda i: (0, 0)),
            pl.BlockSpec((1, 128), lambda i: (0, 0)),
        ],
        out_specs=pl.BlockSpec((bp, 128), lambda i: (i, 0)),
        out_shape=jax.ShapeDtypeStruct((n_tok // 2, 128), jnp.float32),
        compiler_params=pltpu.CompilerParams(
            dimension_semantics=("arbitrary",),
        ),
    )(g2, wfull, bias2)


NSLICE = 4


def kernel(x, table, W, b):
    bs, seq, k = x.shape
    ke = k * EMBED
    v = table.shape[0]
    tablep = jnp.pad(table, ((0, 0), (0, EMBED))).reshape(2 * v, EMBED)
    x2 = (x * 2).reshape(bs, seq * k)
    z = jnp.zeros((ke, EMBED), jnp.float32)
    wfull = jnp.concatenate(
        [jnp.concatenate([W, z], axis=1), jnp.concatenate([z, W], axis=1)],
        axis=0,
    )
    bias2 = jnp.concatenate([b, b]).reshape(1, 2 * EMBED)

    bsl = bs // NSLICE
    outs = []
    for s in range(NSLICE):
        xs = lax.slice_in_dim(x2, s * bsl, (s + 1) * bsl, axis=0)
        g = _sc_gather(tablep, xs)
        g2 = g.reshape(bsl * seq * k // 2, 2 * EMBED)
        outs.append(_tc_project(g2, wfull, bias2, bsl * seq))
    out2 = jnp.concatenate(outs, axis=0)
    return out2.reshape(bs, seq, EMBED)
